# Initial kernel scaffold; baseline (speedup 1.0000x reference)
import jax
import jax.numpy as jnp
from jax.experimental import pallas as pl


def kernel(inputs):
    raise NotImplementedError("write your pallas kernel here")


# speedup vs baseline: 517.0716x; 517.0716x over previous
import functools

import jax
import jax.numpy as jnp
from jax.experimental import pallas as pl
from jax.experimental.pallas import tpu as pltpu

_W0, _W1, _W2 = 0.2989, 0.5870, 0.1140
_NB = 256


def _gray_kernel(x_ref, p_ref, gray_ref, mn_ref, mx_ref, mn_s, mx_s,
                 *, chunks):
    i = pl.program_id(1)
    rows = x_ref.shape[0]
    p = p_ref[...]
    mns, mxs = [], []
    for k in range(chunks):
        xc = x_ref[:, k * 768:(k + 1) * 768]
        g = jnp.dot(xc, p, preferred_element_type=jnp.float32)
        gray_ref[:, k * 256:(k + 1) * 256] = g
        m = None
        mm = None
        for r in range(rows // 8):
            for c in range(2):
                v = g[r * 8:(r + 1) * 8, c * 128:(c + 1) * 128]
                m = v if m is None else jnp.minimum(m, v)
                mm = v if mm is None else jnp.maximum(mm, v)
        mns.append(m)
        mxs.append(mm)
    bmn = functools.reduce(jnp.minimum, mns)
    bmx = functools.reduce(jnp.maximum, mxs)

    @pl.when(i == 0)
    def _():
        mn_s[...] = bmn
        mx_s[...] = bmx

    @pl.when(i != 0)
    def _():
        mn_s[...] = jnp.minimum(mn_s[...], bmn)
        mx_s[...] = jnp.maximum(mx_s[...], bmx)

    mn_ref[...] = mn_s[...][None]
    mx_ref[...] = mx_s[...][None]


def _hist_kernel(g_ref, mn_ref, mx_ref, hist_ref, acc, idx_s, *, nsteps,
                 slabs):
    j = pl.program_id(1)
    vmin = jnp.min(mn_ref[...])
    vmax = jnp.max(mx_ref[...])
    scale = jnp.where(vmax > vmin, float(_NB) / (vmax - vmin), 0.0)
    g = g_ref[...]
    idx_s[...] = jnp.clip(jnp.floor((g - vmin) * scale), 0.0, 255.0)

    @pl.when(j == 0)
    def _():
        acc[...] = jnp.zeros_like(acc)

    def body(b, carry):
        bf = b.astype(jnp.float32)
        tot = None
        for t in range(slabs):
            sl = idx_s[t * 8:(t + 1) * 8, :]
            m = jnp.where(sl == bf, 1.0, 0.0)
            tot = m if tot is None else tot + m
        acc[b] = acc[b] + tot
        return carry

    jax.lax.fori_loop(0, _NB, body, 0)

    @pl.when(j == nsteps - 1)
    def _():
        for b in range(_NB):
            v = acc[b]
            srow = None
            for c in range(v.shape[1] // 128):
                vc = v[:, c * 128:(c + 1) * 128]
                srow = vc if srow is None else srow + vc
            hist_ref[:, b, :] = jnp.sum(srow, axis=0, keepdims=True)


def _otsu_threshold(hist_ref, vmin, vmax):
    h2 = hist_ref[0] + hist_ref[1]
    ht = h2.T
    h = jnp.sum(ht, axis=0, keepdims=True)
    step = (vmax - vmin) / float(_NB)
    lane_i = jax.lax.broadcasted_iota(jnp.int32, (1, _NB), 1)
    lane_f = lane_i.astype(jnp.float32)
    centers = vmin + (lane_f + 0.5) * step

    def prefix(v):
        w = v
        k = 1
        while k < _NB:
            sh = jnp.roll(w, k, axis=1)
            w = w + jnp.where(lane_i >= k, sh, 0.0)
            k *= 2
        return w

    def suffix(v):
        w = v
        k = 1
        while k < _NB:
            sh = jnp.roll(w, -k, axis=1)
            w = w + jnp.where(lane_i < _NB - k, sh, 0.0)
            k *= 2
        return w

    w1 = prefix(h)
    total = w1[:, _NB - 1:_NB]
    w2 = (total - w1) + h
    hc = h * centers
    s1 = prefix(hc)
    s2 = suffix(hc)
    m1 = s1 / jnp.maximum(w1, 1.0)
    m2 = s2 / jnp.maximum(w2, 1.0)
    d = m1[:, :_NB - 1] - m2[:, 1:]
    var12 = (w1[:, :_NB - 1] * w2[:, 1:]) * (d * d)
    idx = jnp.argmax(var12, axis=1)
    sel = jnp.where(lane_i[:, :_NB - 1] == idx[0], centers[:, :_NB - 1], 0.0)
    return jnp.sum(sel)


def _bin_kernel(g_ref, hist_ref, mn_ref, mx_ref, e_ref, out_ref):
    vmin = jnp.min(mn_ref[...])
    vmax = jnp.max(mx_ref[...])
    thresh = _otsu_threshold(hist_ref, vmin, vmax)
    g = g_ref[...]
    b01 = jnp.where(g > thresh, 1.0, 0.0).astype(jnp.bfloat16)
    e = e_ref[...]
    for k in range(g.shape[1] // 256):
        out_ref[:, k * 768:(k + 1) * 768] = jnp.dot(
            b01[:, k * 256:(k + 1) * 256], e,
            preferred_element_type=jnp.float32)


def kernel(inputs):
    b, hh, w, c = inputs.shape
    nr = b * hh
    ll = w * c
    wcomp = w
    f32 = jnp.float32
    x2d = inputs.reshape(nr, ll)

    ii = jnp.arange(768)[:, None]
    jj = jnp.arange(256)[None, :]
    wvals = jnp.array([_W0, _W1, _W2], f32)
    pmat = jnp.where(ii // 3 == jj, wvals[ii % 3], 0.0).astype(f32)
    emat = (jj.T == (ii.T // 3)).astype(jnp.bfloat16)

    cparams = pltpu.CompilerParams(
        dimension_semantics=("parallel", "arbitrary"),
        vmem_limit_bytes=48 * 1024 * 1024,
    )

    ra = 128
    na = nr // (2 * ra)
    gray, mn, mx = pl.pallas_call(
        functools.partial(_gray_kernel, chunks=ll // 768),
        grid=(2, na),
        in_specs=[
            pl.BlockSpec((ra, ll), lambda cc, i: (cc * na + i, 0)),
            pl.BlockSpec((768, 256), lambda cc, i: (0, 0)),
        ],
        out_specs=[
            pl.BlockSpec((ra, wcomp), lambda cc, i: (cc * na + i, 0)),
            pl.BlockSpec((1, 8, 128), lambda cc, i: (cc, 0, 0)),
            pl.BlockSpec((1, 8, 128), lambda cc, i: (cc, 0, 0)),
        ],
        out_shape=[
            jax.ShapeDtypeStruct((nr, wcomp), f32),
            jax.ShapeDtypeStruct((2, 8, 128), f32),
            jax.ShapeDtypeStruct((2, 8, 128), f32),
        ],
        scratch_shapes=[pltpu.VMEM((8, 128), f32), pltpu.VMEM((8, 128), f32)],
        compiler_params=cparams,
        name="gray_minmax",
    )(x2d, pmat)

    rb = 256
    nb = nr // (2 * rb)
    hist = pl.pallas_call(
        functools.partial(_hist_kernel, nsteps=nb, slabs=rb // 8),
        grid=(2, nb),
        in_specs=[
            pl.BlockSpec((rb, wcomp), lambda cc, j: (cc * nb + j, 0)),
            pl.BlockSpec((2, 8, 128), lambda cc, j: (0, 0, 0)),
            pl.BlockSpec((2, 8, 128), lambda cc, j: (0, 0, 0)),
        ],
        out_specs=pl.BlockSpec((1, _NB, 128), lambda cc, j: (cc, 0, 0)),
        out_shape=jax.ShapeDtypeStruct((2, _NB, 128), f32),
        scratch_shapes=[
            pltpu.VMEM((_NB, 8, wcomp), f32),
            pltpu.VMEM((rb, wcomp), f32),
        ],
        compiler_params=cparams,
        name="gray_hist",
    )(gray, mn, mx)

    rd = 128
    nd = nr // (2 * rd)
    out2d = pl.pallas_call(
        _bin_kernel,
        grid=(2, nd),
        in_specs=[
            pl.BlockSpec((rd, wcomp), lambda cc, i: (cc * nd + i, 0)),
            pl.BlockSpec((2, _NB, 128), lambda cc, i: (0, 0, 0)),
            pl.BlockSpec((2, 8, 128), lambda cc, i: (0, 0, 0)),
            pl.BlockSpec((2, 8, 128), lambda cc, i: (0, 0, 0)),
            pl.BlockSpec((256, 768), lambda cc, i: (0, 0)),
        ],
        out_specs=pl.BlockSpec((rd, ll), lambda cc, i: (cc * nd + i, 0)),
        out_shape=jax.ShapeDtypeStruct((nr, ll), f32),
        compiler_params=cparams,
        name="otsu_binarize",
    )(gray, hist, mn, mx, emat)

    return out2d.reshape(b, hh, w, c)


# speedup vs baseline: 788.7304x; 1.5254x over previous
import functools

import numpy as np

import jax
import jax.numpy as jnp
from jax.experimental import pallas as pl
from jax.experimental.pallas import tpu as pltpu

_W0, _W1, _W2 = 0.2989, 0.5870, 0.1140
_NB = 256


def _bf16_round(v):
    u = np.float32(v).view(np.uint32)
    u = np.uint32((int(u) + 0x7FFF + ((int(u) >> 16) & 1)) & 0xFFFF0000)
    return float(u.view(np.float32))


def _gray_kernel(x_ref, gray_ref, mn_ref, mx_ref, mn_s, mx_s, *, wb):
    i = pl.program_id(1)
    hrows = gray_ref.shape[0]
    xb = x_ref[...].astype(jnp.bfloat16).astype(jnp.float32)
    g = ((xb[0:hrows] * wb[0] + xb[hrows:2 * hrows] * wb[1])
         + xb[2 * hrows:3 * hrows] * wb[2])
    gray_ref[...] = g
    m = None
    mm = None
    for r in range(hrows // 8):
        for c in range(g.shape[1] // 128):
            v = g[r * 8:(r + 1) * 8, c * 128:(c + 1) * 128]
            m = v if m is None else jnp.minimum(m, v)
            mm = v if mm is None else jnp.maximum(mm, v)

    @pl.when(i == 0)
    def _():
        mn_s[...] = m
        mx_s[...] = mm

    @pl.when(i != 0)
    def _():
        mn_s[...] = jnp.minimum(mn_s[...], m)
        mx_s[...] = jnp.maximum(mx_s[...], mm)

    mn_ref[...] = mn_s[...][None]
    mx_ref[...] = mx_s[...][None]


def _hist_kernel(g_ref, mn_ref, mx_ref, hist_ref, acc, idx_s, *, nsteps,
                 slabs):
    j = pl.program_id(1)
    vmin = jnp.min(mn_ref[...])
    vmax = jnp.max(mx_ref[...])
    scale = jnp.where(vmax > vmin, float(_NB) / (vmax - vmin), 0.0)
    g = g_ref[...]
    idx_s[...] = jnp.clip(jnp.floor((g - vmin) * scale), 0.0, 255.0)

    @pl.when(j == 0)
    def _():
        acc[...] = jnp.zeros_like(acc)

    def body(b, carry):
        bf = b.astype(jnp.float32)
        tot = None
        for t in range(slabs):
            sl = idx_s[t * 8:(t + 1) * 8, :]
            m = jnp.where(sl == bf, 1.0, 0.0)
            tot = m if tot is None else tot + m
        acc[b] = acc[b] + tot
        return carry

    jax.lax.fori_loop(0, _NB, body, 0)

    @pl.when(j == nsteps - 1)
    def _():
        for b in range(_NB):
            v = acc[b]
            srow = None
            for c in range(v.shape[1] // 128):
                vc = v[:, c * 128:(c + 1) * 128]
                srow = vc if srow is None else srow + vc
            hist_ref[:, b, :] = jnp.sum(srow, axis=0, keepdims=True)


def _otsu_threshold(hist_ref, vmin, vmax):
    h2 = hist_ref[0] + hist_ref[1]
    ht = h2.T
    h = jnp.sum(ht, axis=0, keepdims=True)
    step = (vmax - vmin) / float(_NB)
    lane_i = jax.lax.broadcasted_iota(jnp.int32, (1, _NB), 1)
    lane_f = lane_i.astype(jnp.float32)
    centers = vmin + (lane_f + 0.5) * step

    def prefix(v):
        w = v
        k = 1
        while k < _NB:
            sh = jnp.roll(w, k, axis=1)
            w = w + jnp.where(lane_i >= k, sh, 0.0)
            k *= 2
        return w

    def suffix(v):
        w = v
        k = 1
        while k < _NB:
            sh = jnp.roll(w, -k, axis=1)
            w = w + jnp.where(lane_i < _NB - k, sh, 0.0)
            k *= 2
        return w

    w1 = prefix(h)
    total = w1[:, _NB - 1:_NB]
    w2 = (total - w1) + h
    hc = h * centers
    s1 = prefix(hc)
    s2 = suffix(hc)
    m1 = s1 / jnp.maximum(w1, 1.0)
    m2 = s2 / jnp.maximum(w2, 1.0)
    d = m1[:, :_NB - 1] - m2[:, 1:]
    var12 = (w1[:, :_NB - 1] * w2[:, 1:]) * (d * d)
    idx = jnp.argmax(var12, axis=1)
    sel = jnp.where(lane_i[:, :_NB - 1] == idx[0], centers[:, :_NB - 1], 0.0)
    return jnp.sum(sel)


def _bin_kernel(g_ref, hist_ref, mn_ref, mx_ref, out_ref):
    vmin = jnp.min(mn_ref[...])
    vmax = jnp.max(mx_ref[...])
    thresh = _otsu_threshold(hist_ref, vmin, vmax)
    b01 = jnp.where(g_ref[...] > thresh, 1.0, 0.0)
    hrows = g_ref.shape[0]
    out_ref[0:hrows] = b01
    out_ref[hrows:2 * hrows] = b01
    out_ref[2 * hrows:3 * hrows] = b01


def kernel(inputs):
    b, hh, w, c = inputs.shape
    nr = b * hh
    f32 = jnp.float32
    x_p = inputs.transpose(0, 3, 1, 2).reshape(b * c * hh, w)

    wb = tuple(_bf16_round(v) for v in (_W0, _W1, _W2))

    cparams = pltpu.CompilerParams(
        dimension_semantics=("parallel", "arbitrary"),
        vmem_limit_bytes=48 * 1024 * 1024,
    )

    na = b // 2
    gray, mn, mx = pl.pallas_call(
        functools.partial(_gray_kernel, wb=wb),
        grid=(2, na),
        in_specs=[
            pl.BlockSpec((c * hh, w), lambda cc, i: (cc * na + i, 0)),
        ],
        out_specs=[
            pl.BlockSpec((hh, w), lambda cc, i: (cc * na + i, 0)),
            pl.BlockSpec((1, 8, 128), lambda cc, i: (cc, 0, 0)),
            pl.BlockSpec((1, 8, 128), lambda cc, i: (cc, 0, 0)),
        ],
        out_shape=[
            jax.ShapeDtypeStruct((nr, w), f32),
            jax.ShapeDtypeStruct((2, 8, 128), f32),
            jax.ShapeDtypeStruct((2, 8, 128), f32),
        ],
        scratch_shapes=[pltpu.VMEM((8, 128), f32), pltpu.VMEM((8, 128), f32)],
        compiler_params=cparams,
        name="gray_minmax",
    )(x_p)

    rb = 256
    nb = nr // (2 * rb)
    hist = pl.pallas_call(
        functools.partial(_hist_kernel, nsteps=nb, slabs=rb // 8),
        grid=(2, nb),
        in_specs=[
            pl.BlockSpec((rb, w), lambda cc, j: (cc * nb + j, 0)),
            pl.BlockSpec((2, 8, 128), lambda cc, j: (0, 0, 0)),
            pl.BlockSpec((2, 8, 128), lambda cc, j: (0, 0, 0)),
        ],
        out_specs=pl.BlockSpec((1, _NB, 128), lambda cc, j: (cc, 0, 0)),
        out_shape=jax.ShapeDtypeStruct((2, _NB, 128), f32),
        scratch_shapes=[
            pltpu.VMEM((_NB, 8, w), f32),
            pltpu.VMEM((rb, w), f32),
        ],
        compiler_params=cparams,
        name="gray_hist",
    )(gray, mn, mx)

    nd = b // 2
    out_p = pl.pallas_call(
        _bin_kernel,
        grid=(2, nd),
        in_specs=[
            pl.BlockSpec((hh, w), lambda cc, i: (cc * nd + i, 0)),
            pl.BlockSpec((2, _NB, 128), lambda cc, i: (0, 0, 0)),
            pl.BlockSpec((2, 8, 128), lambda cc, i: (0, 0, 0)),
            pl.BlockSpec((2, 8, 128), lambda cc, i: (0, 0, 0)),
        ],
        out_specs=pl.BlockSpec((c * hh, w), lambda cc, i: (cc * nd + i, 0)),
        out_shape=jax.ShapeDtypeStruct((b * c * hh, w), f32),
        compiler_params=cparams,
        name="otsu_binarize",
    )(gray, hist, mn, mx)

    return out_p.reshape(b, c, hh, w).transpose(0, 2, 3, 1)


# speedup vs baseline: 1329.4022x; 1.6855x over previous
import functools

import numpy as np

import jax
import jax.numpy as jnp
from jax.experimental import pallas as pl
from jax.experimental.pallas import tpu as pltpu

_W0, _W1, _W2 = 0.2989, 0.5870, 0.1140
_NB = 256


def _bf16_round(v):
    u = np.float32(v).view(np.uint32)
    u = np.uint32((int(u) + 0x7FFF + ((int(u) >> 16) & 1)) & 0xFFFF0000)
    return float(u.view(np.float32))


def _gray_kernel(x_ref, gray_ref, mn_ref, mx_ref, mn_s, mx_s, *, wb):
    i = pl.program_id(1)
    hrows = gray_ref.shape[0]
    xb = x_ref[...].astype(jnp.bfloat16).astype(jnp.float32)
    g = ((xb[0:hrows] * wb[0] + xb[hrows:2 * hrows] * wb[1])
         + xb[2 * hrows:3 * hrows] * wb[2])
    gray_ref[...] = g
    m = None
    mm = None
    for r in range(hrows // 8):
        for c in range(g.shape[1] // 128):
            v = g[r * 8:(r + 1) * 8, c * 128:(c + 1) * 128]
            m = v if m is None else jnp.minimum(m, v)
            mm = v if mm is None else jnp.maximum(mm, v)

    @pl.when(i == 0)
    def _():
        mn_s[...] = m
        mx_s[...] = mm

    @pl.when(i != 0)
    def _():
        mn_s[...] = jnp.minimum(mn_s[...], m)
        mx_s[...] = jnp.maximum(mx_s[...], mm)

    mn_ref[...] = mn_s[...][None]
    mx_ref[...] = mx_s[...][None]


def _hist_kernel(g_ref, mn_ref, mx_ref, hist_ref, acc, idx_s, *, nsteps,
                 slabs):
    j = pl.program_id(1)
    vmin = jnp.min(mn_ref[...])
    vmax = jnp.max(mx_ref[...])
    scale = jnp.where(vmax > vmin, float(_NB) / (vmax - vmin), 0.0)
    g = g_ref[...]
    idx_s[...] = jnp.clip(jnp.floor((g - vmin) * scale), 0.0,
                          255.0).astype(jnp.bfloat16)

    @pl.when(j == 0)
    def _():
        acc[...] = jnp.zeros_like(acc)

    one = jnp.bfloat16(1.0)
    zero = jnp.bfloat16(0.0)

    def body(b, carry):
        bf = b.astype(jnp.bfloat16)
        tot = None
        for t in range(slabs):
            sl = idx_s[t * 16:(t + 1) * 16, :]
            m = jnp.where(sl == bf, one, zero)
            tot = m if tot is None else tot + m
        acc[b] = acc[b] + tot
        return carry

    jax.lax.fori_loop(0, _NB, body, 0)

    @pl.when(j == nsteps - 1)
    def _():
        for b in range(_NB):
            v = acc[b].astype(jnp.float32)
            srow = None
            for c in range(v.shape[1] // 128):
                vc = v[:, c * 128:(c + 1) * 128]
                srow = vc if srow is None else srow + vc
            hist_ref[:, b, :] = jnp.sum(srow, axis=0, keepdims=True)


def _otsu_threshold(hist_ref, vmin, vmax):
    h2 = hist_ref[0] + hist_ref[1]
    ht = h2.T
    h = jnp.sum(ht, axis=0, keepdims=True)
    step = (vmax - vmin) / float(_NB)
    lane_i = jax.lax.broadcasted_iota(jnp.int32, (1, _NB), 1)
    lane_f = lane_i.astype(jnp.float32)
    centers = vmin + (lane_f + 0.5) * step

    def prefix(v):
        w = v
        k = 1
        while k < _NB:
            sh = jnp.roll(w, k, axis=1)
            w = w + jnp.where(lane_i >= k, sh, 0.0)
            k *= 2
        return w

    def suffix(v):
        w = v
        k = 1
        while k < _NB:
            sh = jnp.roll(w, -k, axis=1)
            w = w + jnp.where(lane_i < _NB - k, sh, 0.0)
            k *= 2
        return w

    w1 = prefix(h)
    total = w1[:, _NB - 1:_NB]
    w2 = (total - w1) + h
    hc = h * centers
    s1 = prefix(hc)
    s2 = suffix(hc)
    m1 = s1 / jnp.maximum(w1, 1.0)
    m2 = s2 / jnp.maximum(w2, 1.0)
    d = m1[:, :_NB - 1] - m2[:, 1:]
    var12 = (w1[:, :_NB - 1] * w2[:, 1:]) * (d * d)
    idx = jnp.argmax(var12, axis=1)
    sel = jnp.where(lane_i[:, :_NB - 1] == idx[0], centers[:, :_NB - 1], 0.0)
    return jnp.sum(sel)


def _bin_kernel(g_ref, hist_ref, mn_ref, mx_ref, out_ref):
    vmin = jnp.min(mn_ref[...])
    vmax = jnp.max(mx_ref[...])
    thresh = _otsu_threshold(hist_ref, vmin, vmax)
    b01 = jnp.where(g_ref[...] > thresh, 1.0, 0.0)
    hrows = g_ref.shape[0]
    out_ref[0:hrows] = b01
    out_ref[hrows:2 * hrows] = b01
    out_ref[2 * hrows:3 * hrows] = b01


def kernel(inputs):
    b, hh, w, c = inputs.shape
    nr = b * hh
    f32 = jnp.float32
    x_p = inputs.transpose(0, 3, 1, 2).reshape(b * c * hh, w)

    wb = tuple(_bf16_round(v) for v in (_W0, _W1, _W2))

    cparams = pltpu.CompilerParams(
        dimension_semantics=("parallel", "arbitrary"),
        vmem_limit_bytes=48 * 1024 * 1024,
    )

    na = b // 2
    gray, mn, mx = pl.pallas_call(
        functools.partial(_gray_kernel, wb=wb),
        grid=(2, na),
        in_specs=[
            pl.BlockSpec((c * hh, w), lambda cc, i: (cc * na + i, 0)),
        ],
        out_specs=[
            pl.BlockSpec((hh, w), lambda cc, i: (cc * na + i, 0)),
            pl.BlockSpec((1, 8, 128), lambda cc, i: (cc, 0, 0)),
            pl.BlockSpec((1, 8, 128), lambda cc, i: (cc, 0, 0)),
        ],
        out_shape=[
            jax.ShapeDtypeStruct((nr, w), f32),
            jax.ShapeDtypeStruct((2, 8, 128), f32),
            jax.ShapeDtypeStruct((2, 8, 128), f32),
        ],
        scratch_shapes=[pltpu.VMEM((8, 128), f32), pltpu.VMEM((8, 128), f32)],
        compiler_params=cparams,
        name="gray_minmax",
    )(x_p)

    rb = 256
    nb = nr // (2 * rb)
    hist = pl.pallas_call(
        functools.partial(_hist_kernel, nsteps=nb, slabs=rb // 16),
        grid=(2, nb),
        in_specs=[
            pl.BlockSpec((rb, w), lambda cc, j: (cc * nb + j, 0)),
            pl.BlockSpec((2, 8, 128), lambda cc, j: (0, 0, 0)),
            pl.BlockSpec((2, 8, 128), lambda cc, j: (0, 0, 0)),
        ],
        out_specs=pl.BlockSpec((1, _NB, 128), lambda cc, j: (cc, 0, 0)),
        out_shape=jax.ShapeDtypeStruct((2, _NB, 128), f32),
        scratch_shapes=[
            pltpu.VMEM((_NB, 16, w), jnp.bfloat16),
            pltpu.VMEM((rb, w), jnp.bfloat16),
        ],
        compiler_params=cparams,
        name="gray_hist",
    )(gray, mn, mx)

    nd = b // 2
    out_p = pl.pallas_call(
        _bin_kernel,
        grid=(2, nd),
        in_specs=[
            pl.BlockSpec((hh, w), lambda cc, i: (cc * nd + i, 0)),
            pl.BlockSpec((2, _NB, 128), lambda cc, i: (0, 0, 0)),
            pl.BlockSpec((2, 8, 128), lambda cc, i: (0, 0, 0)),
            pl.BlockSpec((2, 8, 128), lambda cc, i: (0, 0, 0)),
        ],
        out_specs=pl.BlockSpec((c * hh, w), lambda cc, i: (cc * nd + i, 0)),
        out_shape=jax.ShapeDtypeStruct((b * c * hh, w), f32),
        compiler_params=cparams,
        name="otsu_binarize",
    )(gray, hist, mn, mx)

    return out_p.reshape(b, c, hh, w).transpose(0, 2, 3, 1)


# speedup vs baseline: 1384.3927x; 1.0414x over previous
import functools

import numpy as np

import jax
import jax.numpy as jnp
from jax.experimental import pallas as pl
from jax.experimental.pallas import tpu as pltpu

_W0, _W1, _W2 = 0.2989, 0.5870, 0.1140
_NB = 256


def _bf16_round(v):
    u = np.float32(v).view(np.uint32)
    u = np.uint32((int(u) + 0x7FFF + ((int(u) >> 16) & 1)) & 0xFFFF0000)
    return float(u.view(np.float32))


def _gray_kernel(x_ref, gray_ref, mn_ref, mx_ref, mn_s, mx_s, *, wb):
    i = pl.program_id(1)
    hrows = gray_ref.shape[0]
    xb = x_ref[...].astype(jnp.bfloat16).astype(jnp.float32)
    g = ((xb[0:hrows] * wb[0] + xb[hrows:2 * hrows] * wb[1])
         + xb[2 * hrows:3 * hrows] * wb[2])
    gray_ref[...] = g
    m = None
    mm = None
    for r in range(hrows // 8):
        for c in range(g.shape[1] // 128):
            v = g[r * 8:(r + 1) * 8, c * 128:(c + 1) * 128]
            m = v if m is None else jnp.minimum(m, v)
            mm = v if mm is None else jnp.maximum(mm, v)

    @pl.when(i == 0)
    def _():
        mn_s[...] = m
        mx_s[...] = mm

    @pl.when(i != 0)
    def _():
        mn_s[...] = jnp.minimum(mn_s[...], m)
        mx_s[...] = jnp.maximum(mx_s[...], mm)

    mn_ref[...] = mn_s[...][None]
    mx_ref[...] = mx_s[...][None]


def _hist_kernel(g_ref, mn_ref, mx_ref, hist_ref, acc, q_s, r_s, *, nsteps,
                 slabs):
    j = pl.program_id(1)
    vmin = jnp.min(mn_ref[...])
    vmax = jnp.max(mx_ref[...])
    scale = jnp.where(vmax > vmin, float(_NB) / (vmax - vmin), 0.0)
    g = g_ref[...]
    idxf = jnp.clip(jnp.floor((g - vmin) * scale), 0.0, 255.0)
    qf = jnp.floor(idxf * 0.25)
    rem = idxf - 4.0 * qf
    q_s[...] = qf.astype(jnp.bfloat16)
    for jj in (1, 2, 3):
        r_s[jj - 1] = jnp.where(rem == float(jj), 1.0, 0.0).astype(
            jnp.bfloat16)

    @pl.when(j == 0)
    def _():
        acc[...] = jnp.zeros_like(acc)

    one = jnp.bfloat16(1.0)
    zero = jnp.bfloat16(0.0)

    def body(q, carry):
        qb = q.astype(jnp.bfloat16)
        tq = t1 = t2 = t3 = None
        for t in range(slabs):
            sl = q_s[t * 16:(t + 1) * 16, :]
            eq = sl == qb
            m = jnp.where(eq, one, zero)
            m1 = jnp.where(eq, r_s[0, t * 16:(t + 1) * 16, :], zero)
            m2 = jnp.where(eq, r_s[1, t * 16:(t + 1) * 16, :], zero)
            m3 = jnp.where(eq, r_s[2, t * 16:(t + 1) * 16, :], zero)
            tq = m if tq is None else tq + m
            t1 = m1 if t1 is None else t1 + m1
            t2 = m2 if t2 is None else t2 + m2
            t3 = m3 if t3 is None else t3 + m3
        acc[4 * q] = acc[4 * q] + tq
        acc[4 * q + 1] = acc[4 * q + 1] + t1
        acc[4 * q + 2] = acc[4 * q + 2] + t2
        acc[4 * q + 3] = acc[4 * q + 3] + t3
        return carry

    jax.lax.fori_loop(0, _NB // 4, body, 0)

    @pl.when(j == nsteps - 1)
    def _():
        for q in range(_NB // 4):
            vq = acc[4 * q].astype(jnp.float32)
            v1 = acc[4 * q + 1].astype(jnp.float32)
            v2 = acc[4 * q + 2].astype(jnp.float32)
            v3 = acc[4 * q + 3].astype(jnp.float32)
            v0 = ((vq - v1) - v2) - v3
            for b, v in ((4 * q, v0), (4 * q + 1, v1), (4 * q + 2, v2),
                         (4 * q + 3, v3)):
                srow = None
                for c in range(v.shape[1] // 128):
                    vc = v[:, c * 128:(c + 1) * 128]
                    srow = vc if srow is None else srow + vc
                hist_ref[:, b, :] = jnp.sum(srow, axis=0, keepdims=True)


def _otsu_threshold(hist_ref, vmin, vmax):
    h2 = hist_ref[0] + hist_ref[1]
    ht = h2.T
    h = jnp.sum(ht, axis=0, keepdims=True)
    step = (vmax - vmin) / float(_NB)
    lane_i = jax.lax.broadcasted_iota(jnp.int32, (1, _NB), 1)
    lane_f = lane_i.astype(jnp.float32)
    centers = vmin + (lane_f + 0.5) * step

    def prefix(v):
        w = v
        k = 1
        while k < _NB:
            sh = jnp.roll(w, k, axis=1)
            w = w + jnp.where(lane_i >= k, sh, 0.0)
            k *= 2
        return w

    def suffix(v):
        w = v
        k = 1
        while k < _NB:
            sh = jnp.roll(w, -k, axis=1)
            w = w + jnp.where(lane_i < _NB - k, sh, 0.0)
            k *= 2
        return w

    w1 = prefix(h)
    total = w1[:, _NB - 1:_NB]
    w2 = (total - w1) + h
    hc = h * centers
    s1 = prefix(hc)
    s2 = suffix(hc)
    m1 = s1 / jnp.maximum(w1, 1.0)
    m2 = s2 / jnp.maximum(w2, 1.0)
    d = m1[:, :_NB - 1] - m2[:, 1:]
    var12 = (w1[:, :_NB - 1] * w2[:, 1:]) * (d * d)
    idx = jnp.argmax(var12, axis=1)
    sel = jnp.where(lane_i[:, :_NB - 1] == idx[0], centers[:, :_NB - 1], 0.0)
    return jnp.sum(sel)


def _bin_kernel(g_ref, hist_ref, mn_ref, mx_ref, out_ref):
    vmin = jnp.min(mn_ref[...])
    vmax = jnp.max(mx_ref[...])
    thresh = _otsu_threshold(hist_ref, vmin, vmax)
    b01 = jnp.where(g_ref[...] > thresh, 1.0, 0.0)
    hrows = g_ref.shape[0]
    out_ref[0:hrows] = b01
    out_ref[hrows:2 * hrows] = b01
    out_ref[2 * hrows:3 * hrows] = b01


def kernel(inputs):
    b, hh, w, c = inputs.shape
    nr = b * hh
    f32 = jnp.float32
    x_p = inputs.transpose(0, 3, 1, 2).reshape(b * c * hh, w)

    wb = tuple(_bf16_round(v) for v in (_W0, _W1, _W2))

    cparams = pltpu.CompilerParams(
        dimension_semantics=("parallel", "arbitrary"),
        vmem_limit_bytes=48 * 1024 * 1024,
    )

    na = b // 2
    gray, mn, mx = pl.pallas_call(
        functools.partial(_gray_kernel, wb=wb),
        grid=(2, na),
        in_specs=[
            pl.BlockSpec((c * hh, w), lambda cc, i: (cc * na + i, 0)),
        ],
        out_specs=[
            pl.BlockSpec((hh, w), lambda cc, i: (cc * na + i, 0)),
            pl.BlockSpec((1, 8, 128), lambda cc, i: (cc, 0, 0)),
            pl.BlockSpec((1, 8, 128), lambda cc, i: (cc, 0, 0)),
        ],
        out_shape=[
            jax.ShapeDtypeStruct((nr, w), f32),
            jax.ShapeDtypeStruct((2, 8, 128), f32),
            jax.ShapeDtypeStruct((2, 8, 128), f32),
        ],
        scratch_shapes=[pltpu.VMEM((8, 128), f32), pltpu.VMEM((8, 128), f32)],
        compiler_params=cparams,
        name="gray_minmax",
    )(x_p)

    rb = 256
    nb = nr // (2 * rb)
    hist = pl.pallas_call(
        functools.partial(_hist_kernel, nsteps=nb, slabs=rb // 16),
        grid=(2, nb),
        in_specs=[
            pl.BlockSpec((rb, w), lambda cc, j: (cc * nb + j, 0)),
            pl.BlockSpec((2, 8, 128), lambda cc, j: (0, 0, 0)),
            pl.BlockSpec((2, 8, 128), lambda cc, j: (0, 0, 0)),
        ],
        out_specs=pl.BlockSpec((1, _NB, 128), lambda cc, j: (cc, 0, 0)),
        out_shape=jax.ShapeDtypeStruct((2, _NB, 128), f32),
        scratch_shapes=[
            pltpu.VMEM((_NB, 16, w), jnp.bfloat16),
            pltpu.VMEM((rb, w), jnp.bfloat16),
            pltpu.VMEM((3, rb, w), jnp.bfloat16),
        ],
        compiler_params=cparams,
        name="gray_hist",
    )(gray, mn, mx)

    nd = b // 2
    out_p = pl.pallas_call(
        _bin_kernel,
        grid=(2, nd),
        in_specs=[
            pl.BlockSpec((hh, w), lambda cc, i: (cc * nd + i, 0)),
            pl.BlockSpec((2, _NB, 128), lambda cc, i: (0, 0, 0)),
            pl.BlockSpec((2, 8, 128), lambda cc, i: (0, 0, 0)),
            pl.BlockSpec((2, 8, 128), lambda cc, i: (0, 0, 0)),
        ],
        out_specs=pl.BlockSpec((c * hh, w), lambda cc, i: (cc * nd + i, 0)),
        out_shape=jax.ShapeDtypeStruct((b * c * hh, w), f32),
        compiler_params=cparams,
        name="otsu_binarize",
    )(gray, hist, mn, mx)

    return out_p.reshape(b, c, hh, w).transpose(0, 2, 3, 1)
